# baseline (speedup 1.0000x reference)
import functools

import jax
import jax.numpy as jnp
from jax import lax
from jax.experimental import pallas as pl
from jax.experimental.pallas import tpu as pltpu
from jax.experimental.pallas import tpu_sc as plsc

VOCAB = 1000000
EMBED = 64
SEQ = 200
BATCH = 4096

_NC = 2
_NS = 16
_NW = _NC * _NS
_BPW = BATCH // _NW
_CB = 4
_CHUNKS = _BPW // _CB
_HALF = SEQ // 2
_ROWS = _CB * SEQ
_NGATHER = _ROWS // _HALF


def _pool_body(idx_hbm, emb_hbm, out_hbm, idx_v, rows_v, stage_v, sem):
    wid = lax.axis_index("s") * _NC + lax.axis_index("c")
    inv = jnp.float32(1.0 / SEQ)

    def chunk_body(g, carry):
        base_elem = wid * _BPW + g * _CB
        pltpu.sync_copy(idx_hbm.at[pl.ds(base_elem * 2, _NGATHER), :], idx_v)
        copies = []
        for j in range(_NGATHER):
            copies.append(
                pltpu.async_copy(
                    emb_hbm.at[idx_v.at[j]],
                    rows_v.at[pl.ds(j * _HALF, _HALF), :],
                    sem,
                )
            )
        for c in copies:
            c.wait()

        for e in range(_CB):
            def row_body(r, acc):
                a0, a1, a2, a3 = acc
                row = e * SEQ + r
                a0 = a0 + rows_v[row, pl.ds(0, 16)]
                a1 = a1 + rows_v[row, pl.ds(16, 16)]
                a2 = a2 + rows_v[row, pl.ds(32, 16)]
                a3 = a3 + rows_v[row, pl.ds(48, 16)]
                return (a0, a1, a2, a3)

            z = jnp.zeros((16,), jnp.float32)
            a0, a1, a2, a3 = lax.fori_loop(0, SEQ, row_body, (z, z, z, z))
            stage_v[e, pl.ds(0, 16)] = a0 * inv
            stage_v[e, pl.ds(16, 16)] = a1 * inv
            stage_v[e, pl.ds(32, 16)] = a2 * inv
            stage_v[e, pl.ds(48, 16)] = a3 * inv

        pltpu.sync_copy(stage_v, out_hbm.at[pl.ds(base_elem, _CB), :])
        return carry

    lax.fori_loop(0, _CHUNKS, chunk_body, 0)


def _sc_pool(idx2, emb_table):
    mesh = plsc.VectorSubcoreMesh(
        core_axis_name="c", subcore_axis_name="s",
        num_cores=_NC, num_subcores=_NS,
    )
    f = pl.kernel(
        _pool_body,
        out_type=jax.ShapeDtypeStruct((BATCH, EMBED), jnp.float32),
        mesh=mesh,
        scratch_types=[
            pltpu.VMEM((_NGATHER, _HALF), jnp.int32),
            pltpu.VMEM((_ROWS, EMBED), jnp.float32),
            pltpu.VMEM((_CB, EMBED), jnp.float32),
            pltpu.SemaphoreType.DMA,
        ],
        compiler_params=pltpu.CompilerParams(use_tc_tiling_on_sc=False),
    )
    return f(idx2, emb_table)


def _mlp_body(p_ref, w1_ref, b1_ref, w2_ref, b2_ref, out_ref):
    p = p_ref[...]
    h = jnp.dot(p, w1_ref[...], preferred_element_type=jnp.float32) + b1_ref[...]
    z = jnp.dot(h, w2_ref[...], preferred_element_type=jnp.float32) + b2_ref[...]
    m = jnp.max(z, axis=-1, keepdims=True)
    e = jnp.exp(z - m)
    out_ref[...] = e / jnp.sum(e, axis=-1, keepdims=True)


def _tc_mlp(pooled, w1t, b1, w2t, b2):
    return pl.pallas_call(
        _mlp_body,
        out_shape=jax.ShapeDtypeStruct((BATCH, 2), jnp.float32),
    )(pooled, w1t, b1, w2t, b2)


@jax.jit
def kernel(x, emb_table, fc1_w, fc1_b, fc2_w, fc2_b):
    idx2 = x.T.reshape(2 * BATCH, _HALF)
    pooled = _sc_pool(idx2, emb_table)
    return _tc_mlp(
        pooled,
        fc1_w.T,
        fc1_b.reshape(1, 10),
        fc2_w.T,
        fc2_b.reshape(1, 2),
    )


# speedup vs baseline: 1.1367x; 1.1367x over previous
import functools

import jax
import jax.numpy as jnp
from jax import lax
from jax.experimental import pallas as pl
from jax.experimental.pallas import tpu as pltpu
from jax.experimental.pallas import tpu_sc as plsc

VOCAB = 1000000
EMBED = 64
SEQ = 200
BATCH = 4096

_NC = 2
_NS = 16
_NW = _NC * _NS
_BPW = BATCH // _NW
_CB = 4
_CHUNKS = _BPW // _CB
_HALF = SEQ // 2
_ROWS = _CB * SEQ
_NGATHER = _ROWS // _HALF


def _pool_body(idx_hbm, emb_hbm, out_hbm, idx0, idx1, rows0, rows1,
               stage_v, sem0, sem1):
    wid = lax.axis_index("s") * _NC + lax.axis_index("c")
    base0 = wid * _BPW
    inv = jnp.float32(1.0 / SEQ)
    z = jnp.zeros((16,), jnp.float32)

    def fire(g, idx_v, rows_v, sem):
        be = base0 + g * _CB
        pltpu.sync_copy(idx_hbm.at[pl.ds(be * 2, _NGATHER), :], idx_v)
        for j in range(_NGATHER):
            pltpu.async_copy(
                emb_hbm.at[idx_v.at[j]],
                rows_v.at[pl.ds(j * _HALF, _HALF), :],
                sem,
            )

    def drain(idx_v, rows_v, sem):
        for j in range(_NGATHER):
            pltpu.make_async_copy(
                emb_hbm.at[idx_v.at[j]],
                rows_v.at[pl.ds(j * _HALF, _HALF), :],
                sem,
            ).wait()

    def accum(g, rows_v):
        for e in range(_CB):
            def row_body(r, acc):
                b0, b1, b2, b3, c0, c1, c2, c3 = acc
                r0 = e * SEQ + r * 4
                b0 = b0 + rows_v[r0, pl.ds(0, 16)]
                b1 = b1 + rows_v[r0, pl.ds(16, 16)]
                b2 = b2 + rows_v[r0, pl.ds(32, 16)]
                b3 = b3 + rows_v[r0, pl.ds(48, 16)]
                c0 = c0 + rows_v[r0 + 1, pl.ds(0, 16)]
                c1 = c1 + rows_v[r0 + 1, pl.ds(16, 16)]
                c2 = c2 + rows_v[r0 + 1, pl.ds(32, 16)]
                c3 = c3 + rows_v[r0 + 1, pl.ds(48, 16)]
                b0 = b0 + rows_v[r0 + 2, pl.ds(0, 16)]
                b1 = b1 + rows_v[r0 + 2, pl.ds(16, 16)]
                b2 = b2 + rows_v[r0 + 2, pl.ds(32, 16)]
                b3 = b3 + rows_v[r0 + 2, pl.ds(48, 16)]
                c0 = c0 + rows_v[r0 + 3, pl.ds(0, 16)]
                c1 = c1 + rows_v[r0 + 3, pl.ds(16, 16)]
                c2 = c2 + rows_v[r0 + 3, pl.ds(32, 16)]
                c3 = c3 + rows_v[r0 + 3, pl.ds(48, 16)]
                return (b0, b1, b2, b3, c0, c1, c2, c3)

            b0, b1, b2, b3, c0, c1, c2, c3 = lax.fori_loop(
                0, SEQ // 4, row_body, (z, z, z, z, z, z, z, z))
            stage_v[e, pl.ds(0, 16)] = (b0 + c0) * inv
            stage_v[e, pl.ds(16, 16)] = (b1 + c1) * inv
            stage_v[e, pl.ds(32, 16)] = (b2 + c2) * inv
            stage_v[e, pl.ds(48, 16)] = (b3 + c3) * inv

        pltpu.sync_copy(stage_v, out_hbm.at[pl.ds(base0 + g * _CB, _CB), :])

    fire(0, idx0, rows0, sem0)

    def body(i, carry):
        g = 2 * i
        fire(g + 1, idx1, rows1, sem1)
        drain(idx0, rows0, sem0)
        accum(g, rows0)
        fire(g + 2, idx0, rows0, sem0)
        drain(idx1, rows1, sem1)
        accum(g + 1, rows1)
        return carry

    lax.fori_loop(0, _CHUNKS // 2 - 1, body, 0)
    g = _CHUNKS - 2
    fire(g + 1, idx1, rows1, sem1)
    drain(idx0, rows0, sem0)
    accum(g, rows0)
    drain(idx1, rows1, sem1)
    accum(g + 1, rows1)


def _sc_pool(idx2, emb_table):
    mesh = plsc.VectorSubcoreMesh(
        core_axis_name="c", subcore_axis_name="s",
        num_cores=_NC, num_subcores=_NS,
    )
    f = pl.kernel(
        _pool_body,
        out_type=jax.ShapeDtypeStruct((BATCH, EMBED), jnp.float32),
        mesh=mesh,
        scratch_types=[
            pltpu.VMEM((_NGATHER, _HALF), jnp.int32),
            pltpu.VMEM((_NGATHER, _HALF), jnp.int32),
            pltpu.VMEM((_ROWS, EMBED), jnp.float32),
            pltpu.VMEM((_ROWS, EMBED), jnp.float32),
            pltpu.VMEM((_CB, EMBED), jnp.float32),
            pltpu.SemaphoreType.DMA,
            pltpu.SemaphoreType.DMA,
        ],
        compiler_params=pltpu.CompilerParams(use_tc_tiling_on_sc=False),
    )
    return f(idx2, emb_table)


def _mlp_body(p_ref, w1_ref, b1_ref, w2_ref, b2_ref, out_ref):
    p = p_ref[...]
    h = jnp.dot(p, w1_ref[...], preferred_element_type=jnp.float32) + b1_ref[...]
    z = jnp.dot(h, w2_ref[...], preferred_element_type=jnp.float32) + b2_ref[...]
    m = jnp.max(z, axis=-1, keepdims=True)
    e = jnp.exp(z - m)
    out_ref[...] = e / jnp.sum(e, axis=-1, keepdims=True)


def _tc_mlp(pooled, w1t, b1, w2t, b2):
    return pl.pallas_call(
        _mlp_body,
        out_shape=jax.ShapeDtypeStruct((BATCH, 2), jnp.float32),
    )(pooled, w1t, b1, w2t, b2)


@jax.jit
def kernel(x, emb_table, fc1_w, fc1_b, fc2_w, fc2_b):
    idx2 = x.T.reshape(2 * BATCH, _HALF)
    pooled = _sc_pool(idx2, emb_table)
    return _tc_mlp(
        pooled,
        fc1_w.T,
        fc1_b.reshape(1, 10),
        fc2_w.T,
        fc2_b.reshape(1, 2),
    )
